# Initial kernel scaffold; baseline (speedup 1.0000x reference)
import jax
import jax.numpy as jnp
from jax.experimental import pallas as pl


def kernel(x):
    raise NotImplementedError("write your pallas kernel here")


# speedup vs baseline: 14.5828x; 14.5828x over previous
import numpy as np
import jax
import jax.numpy as jnp
from jax.experimental import pallas as pl
from jax.experimental.pallas import tpu as pltpu

_OFF = int(np.asarray(1e-4, np.float32).view(np.int32))
_TOP = 0x3F800000
_N_ITERS = (_TOP - 1 - (_OFF + 1)).bit_length()
_GAMMA = 1.0 / 2.2
_MEAN = (0.485, 0.456, 0.406)
_STD = (0.229, 0.224, 0.225)


def _sel_body(x_ref, n_ref, lo_ref, hi_ref):
    bits = jax.lax.bitcast_convert_type(x_ref[0], jnp.int32)
    valid = bits > _OFF
    n = jnp.sum(valid.astype(jnp.int32))
    k2 = jnp.minimum((2 * n) // 100 + 1, n)
    k98 = jnp.minimum((98 * n) // 100 + 1, n)

    def count(m):
        return jnp.sum((valid & (bits <= m)).astype(jnp.int32))

    def body(_, st):
        a2, b2, a9, b9 = st
        m2 = (a2 + b2) >> 1
        m9 = (a9 + b9) >> 1
        c2 = count(m2)
        c9 = count(m9)
        a2n = jnp.where(c2 >= k2, a2, m2 + 1)
        b2n = jnp.where(c2 >= k2, m2, b2)
        a9n = jnp.where(c9 >= k98, a9, m9 + 1)
        b9n = jnp.where(c9 >= k98, m9, b9)
        return (a2n, b2n, a9n, b9n)

    a0 = jnp.int32(_OFF + 1)
    b0 = jnp.int32(_TOP - 1)
    a2, _, a9, _ = jax.lax.fori_loop(0, _N_ITERS, body, (a0, b0, a0, b0))
    n_ref[...] = jnp.broadcast_to(n, (1, 1, 128)).astype(jnp.int32)
    lo_ref[...] = jnp.broadcast_to(a2, (1, 1, 128)).astype(jnp.int32)
    hi_ref[...] = jnp.broadcast_to(a9, (1, 1, 128)).astype(jnp.int32)


def _select(x48):
    out = jax.ShapeDtypeStruct((48, 1, 128), jnp.int32)
    n, lo, hi = pl.pallas_call(
        _sel_body,
        grid=(48,),
        in_specs=[pl.BlockSpec((1, 256, 1024), lambda p: (p, 0, 0))],
        out_specs=[pl.BlockSpec((1, 1, 128), lambda p: (p, 0, 0))] * 3,
        out_shape=[out, out, out],
    )(x48)
    return n[:, 0, 0], lo[:, 0, 0], hi[:, 0, 0]


def _ew_body(lo_ref, sc_ref, mean_ref, std_ref, x_ref, o_ref):
    p = pl.program_id(0)
    lo = lo_ref[p]
    sc = sc_ref[p]
    mean = mean_ref[p]
    std = std_ref[p]
    y = jnp.clip((x_ref[0] - lo) / sc, 0.0, 1.0)
    y = jnp.exp(jnp.log(y) * _GAMMA)
    o_ref[0] = (y - mean) / std


def _elementwise(x48, lo, scale, mean, std):
    return pl.pallas_call(
        _ew_body,
        grid=(48,),
        in_specs=[
            pl.BlockSpec(memory_space=pltpu.SMEM),
            pl.BlockSpec(memory_space=pltpu.SMEM),
            pl.BlockSpec(memory_space=pltpu.SMEM),
            pl.BlockSpec(memory_space=pltpu.SMEM),
            pl.BlockSpec((1, 256, 1024), lambda p: (p, 0, 0)),
        ],
        out_specs=pl.BlockSpec((1, 256, 1024), lambda p: (p, 0, 0)),
        out_shape=jax.ShapeDtypeStruct((48, 256, 1024), jnp.float32),
    )(lo, scale, mean, std, x48)


def kernel(x):
    B, C, H, W = x.shape
    x48 = x.reshape(B * C, 256, (H * W) // 256)
    n, lo_bits, hi_bits = _select(x48)
    lo_f = jax.lax.bitcast_convert_type(lo_bits, jnp.float32)
    hi_f = jax.lax.bitcast_convert_type(hi_bits, jnp.float32)
    enough = n > 100
    min_val = jnp.where(enough, lo_f, jnp.float32(0.0))
    max_val = jnp.where(enough, hi_f, jnp.float32(1.0))
    scale = jnp.maximum(max_val - min_val, 1e-6)
    mean = jnp.tile(jnp.asarray(_MEAN, jnp.float32), B)
    std = jnp.tile(jnp.asarray(_STD, jnp.float32), B)
    out = _elementwise(x48, min_val, scale, mean, std)
    return out.reshape(B, C, H, W)


# speedup vs baseline: 31.4044x; 2.1535x over previous
import functools

import numpy as np
import jax
import jax.numpy as jnp
from jax import lax
from jax.experimental import pallas as pl
from jax.experimental.pallas import tpu as pltpu
from jax.experimental.pallas import tpu_sc as plsc

_OFF = int(np.asarray(1e-4, np.float32).view(np.int32))
_GAMMA = 1.0 / 2.2
_MEAN = (0.485, 0.456, 0.406)
_STD = (0.229, 0.224, 0.225)

_NC, _NS, _L = 2, 16, 16
_NW = _NC * _NS
_PLANES = 48
_ROWS = 512
_W = 512
_TASKS = 96
_TROWS = _ROWS // 2
_CHR = 16
_NCHUNK = _TROWS // _CHR
_UNROLL = 8
_TPW = _TASKS // _NW

_mesh = plsc.VectorSubcoreMesh(core_axis_name="c", subcore_axis_name="s")


def _worker_id():
    return lax.axis_index("s") * _NC + lax.axis_index("c")


def _zero_hist(hist, nwords):
    z = jnp.zeros((_L,), jnp.int32)

    @plsc.parallel_loop(0, nwords // _L, unroll=_UNROLL)
    def _(r):
        hist[pl.ds(r * _L, _L)] = z


def _lane_reduce(hist, res, nbuckets, lane):
    zero = jnp.zeros((_L,), jnp.int32)

    @plsc.parallel_loop(0, nbuckets // _L, unroll=2)
    def _(j):
        bb = (j * _L + lane) * _L
        acc = zero
        for i in range(_L):
            acc = acc + plsc.load_gather(hist, [bb + i])
        res[pl.ds(j * _L, _L)] = acc


def _chunk_loop(x_hbm, row0, buf0, buf1, sem0, sem1, process_vec):
    bufs = (buf0, buf1)
    sems = (sem0, sem1)
    cps = [None, None]
    cps[0] = pltpu.async_copy(x_hbm.at[pl.ds(row0, _CHR)], buf0, sem0)
    for ch in range(_NCHUNK):
        cur = ch & 1
        cps[cur].wait()
        if ch + 1 < _NCHUNK:
            nxt = 1 - cur
            cps[nxt] = pltpu.async_copy(
                x_hbm.at[pl.ds(row0 + (ch + 1) * _CHR, _CHR)], bufs[nxt], sems[nxt])
        buf = bufs[cur]

        @plsc.parallel_loop(0, _CHR * _W // _L, unroll=_UNROLL)
        def _(j):
            r = j // (_W // _L)
            c = j % (_W // _L)
            bits = buf[r, pl.ds(c * _L, _L)]
            process_vec(bits - (_OFF + 1))


def _pass1_body(x_hbm, out_hbm, hist, res, buf0, buf1, sem0, sem1):
    wid = _worker_id()
    lane = lax.broadcasted_iota(jnp.int32, (_L,), 0)
    ones = jnp.ones((_L,), jnp.int32)
    for t3 in range(_TPW):
        task = wid * _TPW + t3
        _zero_hist(hist, 2048 * _L)

        def process(b):
            idx = lax.shift_right_arithmetic(b, 16)
            plsc.addupdate_scatter(hist, [idx * _L + lane], ones, mask=b >= 0)

        _chunk_loop(x_hbm, task * _TROWS, buf0, buf1, sem0, sem1, process)
        _lane_reduce(hist, res, 2048, lane)
        pltpu.sync_copy(res, out_hbm.at[task])


def _make_pass1():
    return functools.partial(
        pl.kernel,
        mesh=_mesh,
        compiler_params=pltpu.CompilerParams(needs_layout_passes=False),
        out_type=jax.ShapeDtypeStruct((_TASKS, 2048), jnp.int32),
        scratch_types=[
            pltpu.VMEM((2048 * _L,), jnp.int32),
            pltpu.VMEM((2048,), jnp.int32),
            pltpu.VMEM((_CHR, _W), jnp.int32),
            pltpu.VMEM((_CHR, _W), jnp.int32),
            pltpu.SemaphoreType.DMA,
            pltpu.SemaphoreType.DMA,
        ],
    )(_pass1_body)


def _refine_body(nbins, shift, sup_shift, x_hbm, prm_hbm, out_hbm,
                 hist, res, prm, buf0, buf1, sem0, sem1):
    wid = _worker_id()
    lane = lax.broadcasted_iota(jnp.int32, (_L,), 0)
    ones = jnp.ones((_L,), jnp.int32)
    pltpu.sync_copy(prm_hbm, prm)
    for t3 in range(_TPW):
        task = wid * _TPW + t3
        plane = task // 2
        plo = plsc.load_gather(prm, [jnp.broadcast_to(plane * 2, (_L,))])
        phi = plsc.load_gather(prm, [jnp.broadcast_to(plane * 2 + 1, (_L,))])
        _zero_hist(hist, 2 * nbins * _L)

        def process(b):
            idx = jnp.bitwise_and(lax.shift_right_arithmetic(b, shift), nbins - 1)
            sup = lax.shift_right_arithmetic(b, sup_shift)
            mlo = sup == plo
            fidx = (jnp.where(mlo, idx, idx + nbins)) * _L + lane
            plsc.addupdate_scatter(hist, [fidx], ones, mask=mlo | (sup == phi))

        _chunk_loop(x_hbm, task * _TROWS, buf0, buf1, sem0, sem1, process)
        _lane_reduce(hist, res, 2 * nbins, lane)
        pltpu.sync_copy(res, out_hbm.at[task])


def _make_refine(nbins, shift):
    sup_shift = shift + int(nbins).bit_length() - 1
    return functools.partial(
        pl.kernel,
        mesh=_mesh,
        compiler_params=pltpu.CompilerParams(needs_layout_passes=False),
        out_type=jax.ShapeDtypeStruct((_TASKS, 2 * nbins), jnp.int32),
        scratch_types=[
            pltpu.VMEM((2 * nbins * _L,), jnp.int32),
            pltpu.VMEM((2 * nbins,), jnp.int32),
            pltpu.VMEM((_TASKS,), jnp.int32),
            pltpu.VMEM((_CHR, _W), jnp.int32),
            pltpu.VMEM((_CHR, _W), jnp.int32),
            pltpu.SemaphoreType.DMA,
            pltpu.SemaphoreType.DMA,
        ],
    )(functools.partial(_refine_body, nbins, shift, sup_shift))


_pass1 = _make_pass1()
_pass2 = _make_refine(2048, 5)
_pass3 = _make_refine(32, 0)


def _rank_step(hist, k):
    cum = jnp.cumsum(hist, axis=-1)
    b = jnp.sum((cum < k[..., None]).astype(jnp.int32), axis=-1)
    below = jnp.take_along_axis(cum, jnp.maximum(b - 1, 0)[..., None], axis=-1)[..., 0]
    below = jnp.where(b > 0, below, 0)
    return b, k - below


def _split_refine(out, pfx):
    o = out.reshape(_PLANES, 2, 2, -1).sum(axis=1)
    alias = (pfx[:, 0] == pfx[:, 1])[:, None]
    hi = jnp.where(alias, o[:, 0], o[:, 1])
    return jnp.stack([o[:, 0], hi], axis=1)


def _ew_body(lo_ref, sc_ref, mean_ref, std_ref, x_ref, o_ref):
    p = pl.program_id(0)
    y = jnp.clip((x_ref[...] - lo_ref[p]) / sc_ref[p], 0.0, 1.0)
    y = jnp.exp(jnp.log(y) * _GAMMA)
    o_ref[...] = (y - mean_ref[p]) / std_ref[p]


def _elementwise(x2, lo, scale, mean, std):
    return pl.pallas_call(
        _ew_body,
        grid=(_PLANES,),
        in_specs=[
            pl.BlockSpec(memory_space=pltpu.SMEM),
            pl.BlockSpec(memory_space=pltpu.SMEM),
            pl.BlockSpec(memory_space=pltpu.SMEM),
            pl.BlockSpec(memory_space=pltpu.SMEM),
            pl.BlockSpec((_ROWS, _W), lambda p: (p, 0)),
        ],
        out_specs=pl.BlockSpec((_ROWS, _W), lambda p: (p, 0)),
        out_shape=jax.ShapeDtypeStruct((_PLANES * _ROWS, _W), jnp.float32),
    )(lo, scale, mean, std, x2)


def kernel(x):
    B, C, H, W = x.shape
    x2 = x.reshape(B * C * H, W)
    xi = lax.bitcast_convert_type(x2, jnp.int32)

    h1 = _pass1(xi)
    H1 = h1.reshape(_PLANES, 2, 2048).sum(axis=1)
    n = H1.sum(axis=-1)
    k2 = jnp.minimum((2 * n) // 100 + 1, n)
    k98 = jnp.minimum((98 * n) // 100 + 1, n)
    k = jnp.stack([k2, k98], axis=-1)

    b1, k1 = _rank_step(jnp.broadcast_to(H1[:, None, :], (_PLANES, 2, 2048)), k)
    h2 = _pass2(xi, b1.reshape(-1).astype(jnp.int32))
    H2 = _split_refine(h2, b1)
    b2, k2r = _rank_step(H2, k1)
    p2 = (b1 << 11) | b2
    h3 = _pass3(xi, p2.reshape(-1).astype(jnp.int32))
    H3 = _split_refine(h3, p2)
    b3, _ = _rank_step(H3, k2r)

    bits = (_OFF + 1) + ((p2 << 5) | b3)
    vals = lax.bitcast_convert_type(bits.astype(jnp.int32), jnp.float32)
    enough = n > 100
    min_val = jnp.where(enough, vals[:, 0], jnp.float32(0.0))
    max_val = jnp.where(enough, vals[:, 1], jnp.float32(1.0))
    scale = jnp.maximum(max_val - min_val, 1e-6)
    mean = jnp.tile(jnp.asarray(_MEAN, jnp.float32), B)
    std = jnp.tile(jnp.asarray(_STD, jnp.float32), B)
    out = _elementwise(x2, min_val, scale, mean, std)
    return out.reshape(B, C, H, W)


# speedup vs baseline: 37.5578x; 1.1959x over previous
import functools

import numpy as np
import jax
import jax.numpy as jnp
from jax import lax
from jax.experimental import pallas as pl
from jax.experimental.pallas import tpu as pltpu
from jax.experimental.pallas import tpu_sc as plsc

_OFF = int(np.asarray(1e-4, np.float32).view(np.int32))
_GAMMA = 1.0 / 2.2
_MEAN = (0.485, 0.456, 0.406)
_STD = (0.229, 0.224, 0.225)

_NC, _NS, _L = 2, 16, 16
_NW = _NC * _NS
_PLANES = 48
_ROWS = 512
_W = 512
_TASKS = 96
_TROWS = _ROWS // 2
_CHR = 32
_NCHUNK = _TROWS // _CHR
_UNROLL = 8
_TPW = _TASKS // _NW

_mesh = plsc.VectorSubcoreMesh(core_axis_name="c", subcore_axis_name="s")


def _worker_id():
    return lax.axis_index("s") * _NC + lax.axis_index("c")


def _zero_hist(hist, nwords):
    z = jnp.zeros((_L,), jnp.int32)

    @plsc.parallel_loop(0, nwords // _L, unroll=_UNROLL)
    def _(r):
        hist[pl.ds(r * _L, _L)] = z


def _lane_reduce(hist, res, nbuckets, lane):
    zero = jnp.zeros((_L,), jnp.int32)

    @plsc.parallel_loop(0, nbuckets // _L, unroll=2)
    def _(j):
        bb = (j * _L + lane) * _L
        acc = zero
        for i in range(_L):
            acc = acc + plsc.load_gather(hist, [bb + i])
        res[pl.ds(j * _L, _L)] = acc


def _chunk_loop(x_hbm, row0, buf0, buf1, sem0, sem1, process_vec):
    bufs = (buf0, buf1)
    sems = (sem0, sem1)
    cps = [None, None]
    cps[0] = pltpu.async_copy(x_hbm.at[pl.ds(row0, _CHR)], buf0, sem0)
    for ch in range(_NCHUNK):
        cur = ch & 1
        cps[cur].wait()
        if ch + 1 < _NCHUNK:
            nxt = 1 - cur
            cps[nxt] = pltpu.async_copy(
                x_hbm.at[pl.ds(row0 + (ch + 1) * _CHR, _CHR)], bufs[nxt], sems[nxt])
        buf = bufs[cur]

        @plsc.parallel_loop(0, _CHR * _W // _L, unroll=_UNROLL)
        def _(j):
            r = j // (_W // _L)
            c = j % (_W // _L)
            v = buf[r, pl.ds(c * _L, _L)]
            bits = plsc.bitcast(v, jnp.int32)
            process_vec(bits - (_OFF + 1))


def _pass1_body(x_hbm, out_hbm, hist, res, buf0, buf1, sem0, sem1):
    wid = _worker_id()
    lane = lax.broadcasted_iota(jnp.int32, (_L,), 0)
    ones = jnp.ones((_L,), jnp.int32)
    for t3 in range(_TPW):
        task = wid * _TPW + t3
        _zero_hist(hist, 2048 * _L)

        def process(b):
            fidx = jnp.bitwise_and(lax.shift_right_arithmetic(b, 12), -_L) + lane
            plsc.addupdate_scatter(hist, [fidx], ones, mask=b >= 0)

        _chunk_loop(x_hbm, task * _TROWS, buf0, buf1, sem0, sem1, process)
        _lane_reduce(hist, res, 2048, lane)
        pltpu.sync_copy(res, out_hbm.at[task])


def _make_pass1():
    return functools.partial(
        pl.kernel,
        mesh=_mesh,
        compiler_params=pltpu.CompilerParams(needs_layout_passes=False),
        out_type=jax.ShapeDtypeStruct((_TASKS, 2048), jnp.int32),
        scratch_types=[
            pltpu.VMEM((2048 * _L,), jnp.int32),
            pltpu.VMEM((2048,), jnp.int32),
            pltpu.VMEM((_CHR, _W), jnp.float32),
            pltpu.VMEM((_CHR, _W), jnp.float32),
            pltpu.SemaphoreType.DMA,
            pltpu.SemaphoreType.DMA,
        ],
    )(_pass1_body)


def _refine_body(nbins, shift, sup_shift, x_hbm, prm_hbm, out_hbm,
                 hist, res, prm, buf0, buf1, sem0, sem1):
    wid = _worker_id()
    lane = lax.broadcasted_iota(jnp.int32, (_L,), 0)
    ones = jnp.ones((_L,), jnp.int32)
    pltpu.sync_copy(prm_hbm, prm)
    for t3 in range(_TPW):
        task = wid * _TPW + t3
        plane = task // 2
        plo = plsc.load_gather(prm, [jnp.broadcast_to(plane * 2, (_L,))])
        phi = plsc.load_gather(prm, [jnp.broadcast_to(plane * 2 + 1, (_L,))])
        _zero_hist(hist, 2 * nbins * _L)

        def process(b):
            idx = jnp.bitwise_and(lax.shift_right_arithmetic(b, shift), nbins - 1)
            sup = lax.shift_right_arithmetic(b, sup_shift)
            mlo = sup == plo
            fidx = (jnp.where(mlo, idx, idx + nbins)) * _L + lane
            plsc.addupdate_scatter(hist, [fidx], ones, mask=mlo | (sup == phi))

        _chunk_loop(x_hbm, task * _TROWS, buf0, buf1, sem0, sem1, process)
        _lane_reduce(hist, res, 2 * nbins, lane)
        pltpu.sync_copy(res, out_hbm.at[task])


def _make_refine(nbins, shift):
    sup_shift = shift + int(nbins).bit_length() - 1
    return functools.partial(
        pl.kernel,
        mesh=_mesh,
        compiler_params=pltpu.CompilerParams(needs_layout_passes=False),
        out_type=jax.ShapeDtypeStruct((_TASKS, 2 * nbins), jnp.int32),
        scratch_types=[
            pltpu.VMEM((2 * nbins * _L,), jnp.int32),
            pltpu.VMEM((2 * nbins,), jnp.int32),
            pltpu.VMEM((_TASKS,), jnp.int32),
            pltpu.VMEM((_CHR, _W), jnp.float32),
            pltpu.VMEM((_CHR, _W), jnp.float32),
            pltpu.SemaphoreType.DMA,
            pltpu.SemaphoreType.DMA,
        ],
    )(functools.partial(_refine_body, nbins, shift, sup_shift))


_pass1 = _make_pass1()
_pass2 = _make_refine(2048, 5)
_pass3 = _make_refine(32, 0)


_TRI2048 = np.triu(np.ones((2048, 2048), np.float32))


def _rank_step(hist, k, tri=None):
    if tri is not None:
        nb = hist.shape[-1]
        cum = jnp.dot(hist.reshape(-1, nb).astype(jnp.float32), tri,
                      precision=lax.Precision.HIGHEST).reshape(hist.shape)
        cum = cum.astype(jnp.int32)
    else:
        cum = jnp.cumsum(hist, axis=-1)
    b = jnp.sum((cum < k[..., None]).astype(jnp.int32), axis=-1)
    below = jnp.take_along_axis(cum, jnp.maximum(b - 1, 0)[..., None], axis=-1)[..., 0]
    below = jnp.where(b > 0, below, 0)
    return b, k - below


def _split_refine(out, pfx):
    o = out.reshape(_PLANES, 2, 2, -1).sum(axis=1)
    alias = (pfx[:, 0] == pfx[:, 1])[:, None]
    hi = jnp.where(alias, o[:, 0], o[:, 1])
    return jnp.stack([o[:, 0], hi], axis=1)


def _ew_body(lo_ref, sc_ref, mean_ref, std_ref, x_ref, o_ref):
    p = pl.program_id(0) // 2
    y = jnp.clip((x_ref[...] - lo_ref[p]) / sc_ref[p], 0.0, 1.0)
    y = jnp.exp(jnp.log(y) * _GAMMA)
    o_ref[...] = (y - mean_ref[p]) / std_ref[p]


def _elementwise(x2, lo, scale, mean, std):
    return pl.pallas_call(
        _ew_body,
        grid=(_PLANES * 2,),
        in_specs=[
            pl.BlockSpec(memory_space=pltpu.SMEM),
            pl.BlockSpec(memory_space=pltpu.SMEM),
            pl.BlockSpec(memory_space=pltpu.SMEM),
            pl.BlockSpec(memory_space=pltpu.SMEM),
            pl.BlockSpec((_ROWS // 2, _W), lambda i: (i, 0)),
        ],
        out_specs=pl.BlockSpec((_ROWS // 2, _W), lambda i: (i, 0)),
        out_shape=jax.ShapeDtypeStruct((_PLANES * _ROWS, _W), jnp.float32),
    )(lo, scale, mean, std, x2)


def kernel(x):
    B, C, H, W = x.shape
    x2 = x.reshape(B * C * H, W)

    h1 = _pass1(x2)
    H1 = h1.reshape(_PLANES, 2, 2048).sum(axis=1)
    n = H1.sum(axis=-1)
    k2 = jnp.minimum((2 * n) // 100 + 1, n)
    k98 = jnp.minimum((98 * n) // 100 + 1, n)
    k = jnp.stack([k2, k98], axis=-1)

    b1, k1 = _rank_step(jnp.broadcast_to(H1[:, None, :], (_PLANES, 2, 2048)), k, _TRI2048)
    h2 = _pass2(x2, b1.reshape(-1).astype(jnp.int32))
    H2 = _split_refine(h2, b1)
    b2, k2r = _rank_step(H2, k1, _TRI2048)
    p2 = (b1 << 11) | b2
    h3 = _pass3(x2, p2.reshape(-1).astype(jnp.int32))
    H3 = _split_refine(h3, p2)
    b3, _ = _rank_step(H3, k2r)

    bits = (_OFF + 1) + ((p2 << 5) | b3)
    vals = lax.bitcast_convert_type(bits.astype(jnp.int32), jnp.float32)
    enough = n > 100
    min_val = jnp.where(enough, vals[:, 0], jnp.float32(0.0))
    max_val = jnp.where(enough, vals[:, 1], jnp.float32(1.0))
    scale = jnp.maximum(max_val - min_val, 1e-6)
    mean = jnp.tile(jnp.asarray(_MEAN, jnp.float32), B)
    std = jnp.tile(jnp.asarray(_STD, jnp.float32), B)
    out = _elementwise(x2, min_val, scale, mean, std)
    return out.reshape(B, C, H, W)


# speedup vs baseline: 44.2751x; 1.1789x over previous
import functools

import numpy as np
import jax
import jax.numpy as jnp
from jax import lax
from jax.experimental import pallas as pl
from jax.experimental.pallas import tpu as pltpu
from jax.experimental.pallas import tpu_sc as plsc

_OFF = int(np.asarray(1e-4, np.float32).view(np.int32))
_GAMMA = 1.0 / 2.2
_MEAN = (0.485, 0.456, 0.406)
_STD = (0.229, 0.224, 0.225)

_NC, _NS, _L = 2, 16, 16
_NW = _NC * _NS
_PLANES = 48
_ROWS = 512
_W = 512
_TASKS = 96
_TROWS = _ROWS // 2
_CHR = 32
_NCHUNK = _TROWS // _CHR
_UNROLL = 8
_TPW = _TASKS // _NW

_mesh = plsc.VectorSubcoreMesh(core_axis_name="c", subcore_axis_name="s")


def _worker_id():
    return lax.axis_index("s") * _NC + lax.axis_index("c")


def _zero_hist(hist, nwords):
    z = jnp.zeros((_L,), jnp.int32)

    @plsc.parallel_loop(0, nwords // _L, unroll=_UNROLL)
    def _(r):
        hist[pl.ds(r * _L, _L)] = z


def _lane_reduce(hist, res, nbuckets, lane):
    zero = jnp.zeros((_L,), jnp.int32)

    @plsc.parallel_loop(0, nbuckets // _L, unroll=2)
    def _(j):
        bb = (j * _L + lane) * _L
        acc = zero
        for i in range(_L):
            acc = acc + plsc.load_gather(hist, [bb + i])
        res[pl.ds(j * _L, _L)] = acc


def _chunk_loop(x_hbm, row0, buf0, buf1, sem0, sem1, process_vec):
    bufs = (buf0, buf1)
    sems = (sem0, sem1)
    cps = [None, None]
    cps[0] = pltpu.async_copy(x_hbm.at[pl.ds(row0, _CHR)], buf0, sem0)
    for ch in range(_NCHUNK):
        cur = ch & 1
        cps[cur].wait()
        if ch + 1 < _NCHUNK:
            nxt = 1 - cur
            cps[nxt] = pltpu.async_copy(
                x_hbm.at[pl.ds(row0 + (ch + 1) * _CHR, _CHR)], bufs[nxt], sems[nxt])
        buf = bufs[cur]

        @plsc.parallel_loop(0, _CHR * _W // _L, unroll=_UNROLL)
        def _(j):
            r = j // (_W // _L)
            c = j % (_W // _L)
            v = buf[r, pl.ds(c * _L, _L)]
            bits = plsc.bitcast(v, jnp.int32)
            process_vec(bits - (_OFF + 1))


def _pass1_body(x_hbm, out_hbm, hist, res, buf0, buf1, sem0, sem1):
    wid = _worker_id()
    lane = lax.broadcasted_iota(jnp.int32, (_L,), 0)
    ones = jnp.ones((_L,), jnp.int32)
    for t3 in range(_TPW):
        task = wid * _TPW + t3
        _zero_hist(hist, 2048 * _L)

        def process(b):
            fidx = jnp.bitwise_and(lax.shift_right_arithmetic(b, 12), -_L) + lane
            plsc.addupdate_scatter(hist, [fidx], ones, mask=b >= 0)

        _chunk_loop(x_hbm, task * _TROWS, buf0, buf1, sem0, sem1, process)
        _lane_reduce(hist, res, 2048, lane)
        pltpu.sync_copy(res, out_hbm.at[task])


def _make_pass1():
    return functools.partial(
        pl.kernel,
        mesh=_mesh,
        compiler_params=pltpu.CompilerParams(needs_layout_passes=False),
        out_type=jax.ShapeDtypeStruct((_TASKS, 2048), jnp.int32),
        scratch_types=[
            pltpu.VMEM((2048 * _L,), jnp.int32),
            pltpu.VMEM((2048,), jnp.int32),
            pltpu.VMEM((_CHR, _W), jnp.float32),
            pltpu.VMEM((_CHR, _W), jnp.float32),
            pltpu.SemaphoreType.DMA,
            pltpu.SemaphoreType.DMA,
        ],
    )(_pass1_body)


def _refine_body(nbins, shift, sup_shift, x_hbm, prm_hbm, out_hbm,
                 hist, res, prm, buf0, buf1, sem0, sem1):
    wid = _worker_id()
    lane = lax.broadcasted_iota(jnp.int32, (_L,), 0)
    ones = jnp.ones((_L,), jnp.int32)
    pltpu.sync_copy(prm_hbm, prm)
    for t3 in range(_TPW):
        task = wid * _TPW + t3
        plane = task // 2
        plo = plsc.load_gather(prm, [jnp.broadcast_to(plane * 2, (_L,))])
        phi = plsc.load_gather(prm, [jnp.broadcast_to(plane * 2 + 1, (_L,))])
        _zero_hist(hist, 2 * nbins * _L)

        def process(b):
            idx = jnp.bitwise_and(lax.shift_right_arithmetic(b, shift), nbins - 1)
            sup = lax.shift_right_arithmetic(b, sup_shift)
            mlo = sup == plo
            fidx = (jnp.where(mlo, idx, idx + nbins)) * _L + lane
            plsc.addupdate_scatter(hist, [fidx], ones, mask=mlo | (sup == phi))

        _chunk_loop(x_hbm, task * _TROWS, buf0, buf1, sem0, sem1, process)
        _lane_reduce(hist, res, 2 * nbins, lane)
        pltpu.sync_copy(res, out_hbm.at[task])


def _make_refine(nbins, shift):
    sup_shift = shift + int(nbins).bit_length() - 1
    return functools.partial(
        pl.kernel,
        mesh=_mesh,
        compiler_params=pltpu.CompilerParams(needs_layout_passes=False),
        out_type=jax.ShapeDtypeStruct((_TASKS, 2 * nbins), jnp.int32),
        scratch_types=[
            pltpu.VMEM((2 * nbins * _L,), jnp.int32),
            pltpu.VMEM((2 * nbins,), jnp.int32),
            pltpu.VMEM((_TASKS,), jnp.int32),
            pltpu.VMEM((_CHR, _W), jnp.float32),
            pltpu.VMEM((_CHR, _W), jnp.float32),
            pltpu.SemaphoreType.DMA,
            pltpu.SemaphoreType.DMA,
        ],
    )(functools.partial(_refine_body, nbins, shift, sup_shift))


_pass1 = _make_pass1()
_pass2 = _make_refine(256, 8)
_pass3 = _make_refine(256, 0)


_TRI16 = np.triu(np.ones((16, 16), np.float32))
_TRIS128 = np.triu(np.ones((128, 128), np.float32), k=1)


def _prefix_sum(hist):
    nb = hist.shape[-1]
    ng = nb // 16
    c = hist.astype(jnp.float32).reshape(-1, ng, 16)
    intra = jnp.dot(c.reshape(-1, 16), _TRI16,
                    precision=lax.Precision.HIGHEST).reshape(-1, ng, 16)
    pre = jnp.dot(intra[..., -1], _TRIS128[:ng, :ng],
                  precision=lax.Precision.HIGHEST)
    return (pre[..., None] + intra).reshape(hist.shape).astype(jnp.int32)


def _rank_step(hist, k):
    cum = _prefix_sum(hist)
    b = jnp.sum((cum < k[..., None]).astype(jnp.int32), axis=-1)
    below = jnp.take_along_axis(cum, jnp.maximum(b - 1, 0)[..., None], axis=-1)[..., 0]
    below = jnp.where(b > 0, below, 0)
    return b, k - below


def _split_refine(out, pfx):
    o = out.reshape(_PLANES, 2, 2, -1).sum(axis=1)
    alias = (pfx[:, 0] == pfx[:, 1])[:, None]
    hi = jnp.where(alias, o[:, 0], o[:, 1])
    return jnp.stack([o[:, 0], hi], axis=1)


def _ew_body(lo_ref, sc_ref, mean_ref, std_ref, x_ref, o_ref):
    p = pl.program_id(0)
    y = jnp.clip((x_ref[...] - lo_ref[p]) / sc_ref[p], 0.0, 1.0)
    y = jnp.exp(jnp.log(y) * _GAMMA)
    o_ref[...] = (y - mean_ref[p]) / std_ref[p]


def _elementwise(x2, lo, scale, mean, std):
    return pl.pallas_call(
        _ew_body,
        grid=(_PLANES,),
        in_specs=[
            pl.BlockSpec(memory_space=pltpu.SMEM),
            pl.BlockSpec(memory_space=pltpu.SMEM),
            pl.BlockSpec(memory_space=pltpu.SMEM),
            pl.BlockSpec(memory_space=pltpu.SMEM),
            pl.BlockSpec((_ROWS, _W), lambda i: (i, 0)),
        ],
        out_specs=pl.BlockSpec((_ROWS, _W), lambda i: (i, 0)),
        out_shape=jax.ShapeDtypeStruct((_PLANES * _ROWS, _W), jnp.float32),
    )(lo, scale, mean, std, x2)


def kernel(x):
    B, C, H, W = x.shape
    x2 = x.reshape(B * C * H, W)

    h1 = _pass1(x2)
    H1 = h1.reshape(_PLANES, 2, 2048).sum(axis=1)
    n = H1.sum(axis=-1)
    k2 = jnp.minimum((2 * n) // 100 + 1, n)
    k98 = jnp.minimum((98 * n) // 100 + 1, n)
    k = jnp.stack([k2, k98], axis=-1)

    b1, k1 = _rank_step(jnp.broadcast_to(H1[:, None, :], (_PLANES, 2, 2048)), k)
    h2 = _pass2(x2, b1.reshape(-1).astype(jnp.int32))
    H2 = _split_refine(h2, b1)
    b2, k2r = _rank_step(H2, k1)
    p2 = (b1 << 8) | b2
    h3 = _pass3(x2, p2.reshape(-1).astype(jnp.int32))
    H3 = _split_refine(h3, p2)
    b3, _ = _rank_step(H3, k2r)

    bits = (_OFF + 1) + ((p2 << 8) | b3)
    vals = lax.bitcast_convert_type(bits.astype(jnp.int32), jnp.float32)
    enough = n > 100
    min_val = jnp.where(enough, vals[:, 0], jnp.float32(0.0))
    max_val = jnp.where(enough, vals[:, 1], jnp.float32(1.0))
    scale = jnp.maximum(max_val - min_val, 1e-6)
    mean = jnp.tile(jnp.asarray(_MEAN, jnp.float32), B)
    std = jnp.tile(jnp.asarray(_STD, jnp.float32), B)
    out = _elementwise(x2, min_val, scale, mean, std)
    return out.reshape(B, C, H, W)


# speedup vs baseline: 46.9328x; 1.0600x over previous
import functools

import numpy as np
import jax
import jax.numpy as jnp
from jax import lax
from jax.experimental import pallas as pl
from jax.experimental.pallas import tpu as pltpu
from jax.experimental.pallas import tpu_sc as plsc

_OFF = int(np.asarray(1e-4, np.float32).view(np.int32))
_GAMMA = 1.0 / 2.2
_MEAN = (0.485, 0.456, 0.406)
_STD = (0.229, 0.224, 0.225)

_NC, _NS, _L = 2, 16, 16
_NW = _NC * _NS
_PLANES = 48
_ROWS = 512
_W = 512
_TASKS = 96
_TROWS = _ROWS // 2
_CHR = 32
_NCHUNK = _TROWS // _CHR
_UNROLL = 8
_TPW = _TASKS // _NW

_mesh = plsc.VectorSubcoreMesh(core_axis_name="c", subcore_axis_name="s")


def _worker_id():
    return lax.axis_index("s") * _NC + lax.axis_index("c")


def _zero_hist(hist, nwords):
    z = jnp.zeros((_L,), jnp.int32)

    @plsc.parallel_loop(0, nwords // _L, unroll=_UNROLL)
    def _(r):
        hist[pl.ds(r * _L, _L)] = z


def _lane_reduce(hist, res, nbuckets, lane):
    zero = jnp.zeros((_L,), jnp.int32)

    @plsc.parallel_loop(0, nbuckets // _L, unroll=2)
    def _(j):
        bb = (j * _L + lane) * _L
        acc = zero
        for i in range(_L):
            acc = acc + plsc.load_gather(hist, [bb + i])
        res[pl.ds(j * _L, _L)] = acc


def _chunk_loop(x_hbm, row0, buf0, buf1, sem0, sem1, process_vec):
    bufs = (buf0, buf1)
    sems = (sem0, sem1)
    cps = [None, None]
    cps[0] = pltpu.async_copy(x_hbm.at[pl.ds(row0, _CHR)], buf0, sem0)
    for ch in range(_NCHUNK):
        cur = ch & 1
        cps[cur].wait()
        if ch + 1 < _NCHUNK:
            nxt = 1 - cur
            cps[nxt] = pltpu.async_copy(
                x_hbm.at[pl.ds(row0 + (ch + 1) * _CHR, _CHR)], bufs[nxt], sems[nxt])
        buf = bufs[cur]

        @plsc.parallel_loop(0, _CHR * _W // _L, unroll=_UNROLL)
        def _(j):
            r = j // (_W // _L)
            c = j % (_W // _L)
            v = buf[r, pl.ds(c * _L, _L)]
            bits = plsc.bitcast(v, jnp.int32)
            process_vec(bits - (_OFF + 1))


def _pass1_body(x_hbm, out_hbm, hist, res, buf0, buf1, sem0, sem1):
    wid = _worker_id()
    lane = lax.broadcasted_iota(jnp.int32, (_L,), 0)
    lane16 = lane + _L
    ones = jnp.ones((_L,), jnp.int32)
    for t3 in range(_TPW):
        task = wid * _TPW + t3
        _zero_hist(hist, 2048 * _L)

        def process(b):
            fidx = jnp.maximum(
                jnp.bitwise_and(lax.shift_right_arithmetic(b, 12), -_L), -_L) + lane16
            plsc.addupdate_scatter(hist, [fidx], ones)

        _chunk_loop(x_hbm, task * _TROWS, buf0, buf1, sem0, sem1, process)
        _lane_reduce(hist, res, 2048, lane)
        pltpu.sync_copy(res, out_hbm.at[task])


def _make_pass1():
    return functools.partial(
        pl.kernel,
        mesh=_mesh,
        compiler_params=pltpu.CompilerParams(needs_layout_passes=False),
        out_type=jax.ShapeDtypeStruct((_TASKS, 2048), jnp.int32),
        scratch_types=[
            pltpu.VMEM((2048 * _L,), jnp.int32),
            pltpu.VMEM((2048,), jnp.int32),
            pltpu.VMEM((_CHR, _W), jnp.float32),
            pltpu.VMEM((_CHR, _W), jnp.float32),
            pltpu.SemaphoreType.DMA,
            pltpu.SemaphoreType.DMA,
        ],
    )(_pass1_body)


def _refine_body(nbins, shift, sup_shift, x_hbm, prm_hbm, out_hbm,
                 hist, res, prm, buf0, buf1, sem0, sem1):
    wid = _worker_id()
    lane = lax.broadcasted_iota(jnp.int32, (_L,), 0)
    ones = jnp.ones((_L,), jnp.int32)
    pltpu.sync_copy(prm_hbm, prm)
    for t3 in range(_TPW):
        task = wid * _TPW + t3
        plane = task // 2
        plo = plsc.load_gather(prm, [jnp.broadcast_to(plane * 2, (_L,))])
        phi = plsc.load_gather(prm, [jnp.broadcast_to(plane * 2 + 1, (_L,))])
        _zero_hist(hist, 2 * nbins * _L)

        def process(b):
            idx = jnp.bitwise_and(lax.shift_right_arithmetic(b, shift), nbins - 1)
            sup = lax.shift_right_arithmetic(b, sup_shift)
            mlo = sup == plo
            fidx = (jnp.where(mlo, idx, idx + nbins)) * _L + lane
            plsc.addupdate_scatter(hist, [fidx], ones, mask=mlo | (sup == phi))

        _chunk_loop(x_hbm, task * _TROWS, buf0, buf1, sem0, sem1, process)
        _lane_reduce(hist, res, 2 * nbins, lane)
        pltpu.sync_copy(res, out_hbm.at[task])


def _make_refine(nbins, shift):
    sup_shift = shift + int(nbins).bit_length() - 1
    return functools.partial(
        pl.kernel,
        mesh=_mesh,
        compiler_params=pltpu.CompilerParams(needs_layout_passes=False),
        out_type=jax.ShapeDtypeStruct((_TASKS, 2 * nbins), jnp.int32),
        scratch_types=[
            pltpu.VMEM((2 * nbins * _L,), jnp.int32),
            pltpu.VMEM((2 * nbins,), jnp.int32),
            pltpu.VMEM((_TASKS,), jnp.int32),
            pltpu.VMEM((_CHR, _W), jnp.float32),
            pltpu.VMEM((_CHR, _W), jnp.float32),
            pltpu.SemaphoreType.DMA,
            pltpu.SemaphoreType.DMA,
        ],
    )(functools.partial(_refine_body, nbins, shift, sup_shift))


_pass1 = _make_pass1()
_pass2 = _make_refine(256, 8)
_pass3 = _make_refine(256, 0)


_TRI16 = np.triu(np.ones((16, 16), np.float32))
_TRIS128 = np.triu(np.ones((128, 128), np.float32), k=1)


def _prefix_sum(hist):
    nb = hist.shape[-1]
    ng = nb // 16
    c = hist.astype(jnp.float32).reshape(-1, ng, 16)
    intra = jnp.dot(c.reshape(-1, 16), _TRI16,
                    precision=lax.Precision.HIGHEST).reshape(-1, ng, 16)
    pre = jnp.dot(intra[..., -1], _TRIS128[:ng, :ng],
                  precision=lax.Precision.HIGHEST)
    return (pre[..., None] + intra).reshape(hist.shape).astype(jnp.int32)


def _rank_step(hist, k):
    cum = _prefix_sum(hist)
    b = jnp.sum((cum < k[..., None]).astype(jnp.int32), axis=-1)
    below = jnp.take_along_axis(cum, jnp.maximum(b - 1, 0)[..., None], axis=-1)[..., 0]
    below = jnp.where(b > 0, below, 0)
    return b, k - below


def _rank_step1(hist, k):
    cum = _prefix_sum(hist)
    b = jnp.sum((cum[:, None, :] < k[..., None]).astype(jnp.int32), axis=-1)
    below = jnp.take_along_axis(cum, jnp.maximum(b - 1, 0), axis=-1)
    below = jnp.where(b > 0, below, 0)
    return b, k - below


def _split_refine(out, pfx):
    o = out.reshape(_PLANES, 2, 2, -1).sum(axis=1)
    alias = (pfx[:, 0] == pfx[:, 1])[:, None]
    hi = jnp.where(alias, o[:, 0], o[:, 1])
    return jnp.stack([o[:, 0], hi], axis=1)


def _ew_body(lo_ref, is_ref, a_ref, b_ref, x_ref, o_ref):
    p = pl.program_id(0)
    y = jnp.clip((x_ref[...] - lo_ref[p]) * is_ref[p], 0.0, 1.0)
    y = jnp.exp(jnp.log(y) * _GAMMA)
    o_ref[...] = y * a_ref[p] - b_ref[p]


def _elementwise(x2, lo, scale, mean, std):
    return pl.pallas_call(
        _ew_body,
        grid=(_PLANES,),
        in_specs=[
            pl.BlockSpec(memory_space=pltpu.SMEM),
            pl.BlockSpec(memory_space=pltpu.SMEM),
            pl.BlockSpec(memory_space=pltpu.SMEM),
            pl.BlockSpec(memory_space=pltpu.SMEM),
            pl.BlockSpec((_ROWS, _W), lambda i: (i, 0)),
        ],
        out_specs=pl.BlockSpec((_ROWS, _W), lambda i: (i, 0)),
        out_shape=jax.ShapeDtypeStruct((_PLANES * _ROWS, _W), jnp.float32),
    )(lo, scale, mean, std, x2)


def kernel(x):
    B, C, H, W = x.shape
    x2 = x.reshape(B * C * H, W)

    h1 = _pass1(x2)
    H1 = h1.reshape(_PLANES, 2, 2048).sum(axis=1)
    H1 = H1.at[:, 0].set(0)
    n = H1.sum(axis=-1)
    k2 = jnp.minimum((2 * n) // 100 + 1, n)
    k98 = jnp.minimum((98 * n) // 100 + 1, n)
    k = jnp.stack([k2, k98], axis=-1)

    b1s, k1 = _rank_step1(H1, k)
    b1 = b1s - 1
    h2 = _pass2(x2, b1.reshape(-1).astype(jnp.int32))
    H2 = _split_refine(h2, b1)
    b2, k2r = _rank_step(H2, k1)
    p2 = (b1 << 8) | b2
    h3 = _pass3(x2, p2.reshape(-1).astype(jnp.int32))
    H3 = _split_refine(h3, p2)
    b3, _ = _rank_step(H3, k2r)

    bits = (_OFF + 1) + ((p2 << 8) | b3)
    vals = lax.bitcast_convert_type(bits.astype(jnp.int32), jnp.float32)
    enough = n > 100
    min_val = jnp.where(enough, vals[:, 0], jnp.float32(0.0))
    max_val = jnp.where(enough, vals[:, 1], jnp.float32(1.0))
    inv_scale = 1.0 / jnp.maximum(max_val - min_val, 1e-6)
    inv_std = jnp.tile(1.0 / jnp.asarray(_STD, jnp.float32), B)
    moff = jnp.tile(jnp.asarray(_MEAN, jnp.float32) / jnp.asarray(_STD, jnp.float32), B)
    out = _elementwise(x2, min_val, inv_scale, inv_std, moff)
    return out.reshape(B, C, H, W)
